# baseline (speedup 1.0000x reference)
import functools

import jax
import jax.numpy as jnp
from jax.experimental import pallas as pl

_N = 2048
_H = 8
_HD = 32
_F = 256
_C = 32
_BI = 256


def _proj1_body(x_ref, w_ref, al_ref, ar_ref, gb_ref, el_ref, er_ref):
    bf16 = jnp.bfloat16
    g = jnp.dot(x_ref[...], w_ref[...], preferred_element_type=jnp.float32)
    n = g.shape[0]
    ones = jnp.ones((n, 1), dtype=bf16)
    zeros = jnp.zeros((n, 128 - _HD - 1), dtype=bf16)
    parts = []
    for h in range(_H):
        parts += [g[:, h * _HD:(h + 1) * _HD].astype(bf16), ones, zeros]
    gb_ref[...] = jnp.concatenate(parts, axis=1)
    el_ref[...] = jnp.dot(g, al_ref[...], preferred_element_type=jnp.float32)
    er_ref[...] = jnp.dot(g, ar_ref[...], preferred_element_type=jnp.float32)


def _scores(mask, a, b, c, d):
    return jnp.where(mask, jnp.maximum(a * b, c * d), jnp.bfloat16(0.0))


def _attn1_body(el_ref, ert_ref, gb_ref, adj_ref, w2_ref, a2l_ref, a2r_ref,
                g2b_ref, el2_ref, er2_ref):
    bf16 = jnp.bfloat16
    mask = adj_ref[...]
    el = el_ref[...]
    ert = ert_ref[...]
    ael = jnp.exp(el).astype(bf16)
    cel = jnp.exp(0.2 * el).astype(bf16)
    ber = jnp.exp(ert).astype(bf16)
    der = jnp.exp(0.2 * ert).astype(bf16)
    parts = []
    for h in range(_H):
        p = _scores(mask, ael[:, h:h + 1], ber[h:h + 1, :],
                    cel[:, h:h + 1], der[h:h + 1, :])
        nd = jnp.dot(p, gb_ref[:, h * 128:(h + 1) * 128],
                     preferred_element_type=jnp.float32)
        parts.append(nd[:, :_HD] / nd[:, _HD:_HD + 1])
    hcat = jnp.concatenate(parts, axis=1)
    hact = jnp.where(hcat > 0, hcat, jnp.exp(jnp.minimum(hcat, 0.0)) - 1.0)
    g2 = jnp.dot(hact, w2_ref[...], preferred_element_type=jnp.float32)
    bi = g2.shape[0]
    g2b_ref[...] = jnp.concatenate(
        [g2.astype(bf16), jnp.ones((bi, 1), bf16),
         jnp.zeros((bi, 64 - _C - 1), bf16)], axis=1)
    el2_ref[...] = jnp.dot(g2, a2l_ref[...], preferred_element_type=jnp.float32)
    er2_ref[...] = jnp.dot(g2, a2r_ref[...], preferred_element_type=jnp.float32)


def _attn2_body(el2_ref, er2t_ref, g2b_ref, adj_ref, out_ref):
    bf16 = jnp.bfloat16
    mask = adj_ref[...]
    el2 = el2_ref[...]
    er2 = er2t_ref[...]
    p = _scores(mask, jnp.exp(el2).astype(bf16), jnp.exp(er2).astype(bf16),
                jnp.exp(0.2 * el2).astype(bf16), jnp.exp(0.2 * er2).astype(bf16))
    nd = jnp.dot(p, g2b_ref[...], preferred_element_type=jnp.float32)
    out_ref[...] = nd[:, :_C] / nd[:, _C:_C + 1]


@functools.partial(jax.jit, static_argnames=())
def kernel(x, adj_mat, W1, a1_l, a1_r, W2, a2_l, a2_r):
    f32 = jnp.float32
    adj = adj_mat.reshape(_N, _N)

    eye = jnp.eye(_H, dtype=f32)
    A1l = jnp.kron(eye, a1_l.astype(f32)[:, None])
    A1r = jnp.kron(eye, a1_r.astype(f32)[:, None])

    g1b, el1, er1 = pl.pallas_call(
        _proj1_body,
        out_shape=(
            jax.ShapeDtypeStruct((_N, _H * 128), jnp.bfloat16),
            jax.ShapeDtypeStruct((_N, _H), f32),
            jax.ShapeDtypeStruct((_N, _H), f32),
        ),
    )(x, W1, A1l, A1r)
    er1_t = er1.T

    nblk = _N // _BI
    g2b, el2, er2 = pl.pallas_call(
        _attn1_body,
        grid=(nblk,),
        in_specs=[
            pl.BlockSpec((_BI, _H), lambda i: (i, 0)),
            pl.BlockSpec((_H, _N), lambda i: (0, 0)),
            pl.BlockSpec((_N, _H * 128), lambda i: (0, 0)),
            pl.BlockSpec((_BI, _N), lambda i: (i, 0)),
            pl.BlockSpec((_F, _C), lambda i: (0, 0)),
            pl.BlockSpec((_C, 1), lambda i: (0, 0)),
            pl.BlockSpec((_C, 1), lambda i: (0, 0)),
        ],
        out_specs=(
            pl.BlockSpec((_BI, 64), lambda i: (i, 0)),
            pl.BlockSpec((_BI, 1), lambda i: (i, 0)),
            pl.BlockSpec((_BI, 1), lambda i: (i, 0)),
        ),
        out_shape=(
            jax.ShapeDtypeStruct((_N, 64), jnp.bfloat16),
            jax.ShapeDtypeStruct((_N, 1), f32),
            jax.ShapeDtypeStruct((_N, 1), f32),
        ),
    )(el1, er1_t, g1b, adj, W2.astype(f32), a2_l.astype(f32)[:, None],
      a2_r.astype(f32)[:, None])
    er2_t = er2.reshape(1, _N)

    out = pl.pallas_call(
        _attn2_body,
        grid=(nblk,),
        in_specs=[
            pl.BlockSpec((_BI, 1), lambda i: (i, 0)),
            pl.BlockSpec((1, _N), lambda i: (0, 0)),
            pl.BlockSpec((_N, 64), lambda i: (0, 0)),
            pl.BlockSpec((_BI, _N), lambda i: (i, 0)),
        ],
        out_specs=pl.BlockSpec((_BI, _C), lambda i: (i, 0)),
        out_shape=jax.ShapeDtypeStruct((_N, _C), f32),
    )(el2, er2_t, g2b, adj)
    return out


# speedup vs baseline: 1.3139x; 1.3139x over previous
import functools

import jax
import jax.numpy as jnp
from jax.experimental import pallas as pl
from jax.experimental.pallas import tpu as pltpu

_N = 2048
_H = 8
_HD = 32
_F = 256
_C = 32
_BI = 256
_NBLK = _N // _BI


def _scores(mask, a, b, c, d):
    return jnp.where(mask, jnp.maximum(a * b, c * d), jnp.bfloat16(0.0))


def _body(x_ref, w1_ref, al_ref, ar_ref, adj_ref, w2_ref, a2l_ref, a2r_ref,
          out_ref, gaug, el1s, er1t, g2aug, el2s, er2t):
    i = pl.program_id(0)
    f32 = jnp.float32
    bf16 = jnp.bfloat16

    @pl.when(i == 0)
    def _proj():
        g = jnp.dot(x_ref[...], w1_ref[...], preferred_element_type=f32)
        ones = jnp.ones((_N, 1), dtype=bf16)
        zeros = jnp.zeros((_N, 128 - _HD - 1), dtype=bf16)
        parts = []
        for h in range(_H):
            parts += [g[:, h * _HD:(h + 1) * _HD].astype(bf16), ones, zeros]
        gaug[...] = jnp.concatenate(parts, axis=1)
        el1s[...] = jnp.dot(g, al_ref[...], preferred_element_type=f32)
        er1t[...] = jnp.dot(g, ar_ref[...], preferred_element_type=f32).T

    @pl.when((i >= 1) & (i <= _NBLK))
    def _layer1():
        r0 = (i - 1) * _BI
        mask = adj_ref[...] != 0
        el = el1s[pl.ds(r0, _BI), :]
        ert = er1t[...]
        ael = jnp.exp(el).astype(bf16)
        cel = jnp.exp(0.2 * el).astype(bf16)
        ber = jnp.exp(ert).astype(bf16)
        der = jnp.exp(0.2 * ert).astype(bf16)
        parts = []
        for h in range(_H):
            p = _scores(mask, ael[:, h:h + 1], ber[h:h + 1, :],
                        cel[:, h:h + 1], der[h:h + 1, :])
            nd = jnp.dot(p, gaug[:, h * 128:(h + 1) * 128],
                         preferred_element_type=f32)
            parts.append(nd[:, :_HD] / nd[:, _HD:_HD + 1])
        hcat = jnp.concatenate(parts, axis=1)
        hact = jnp.where(hcat > 0, hcat,
                         jnp.exp(jnp.minimum(hcat, 0.0)) - 1.0)
        g2 = jnp.dot(hact, w2_ref[...], preferred_element_type=f32)
        g2aug[pl.ds(r0, _BI), :] = jnp.concatenate(
            [g2.astype(bf16), jnp.ones((_BI, 1), bf16),
             jnp.zeros((_BI, 64 - _C - 1), bf16)], axis=1)
        el2s[pl.ds(r0, _BI), :] = jnp.dot(g2, a2l_ref[...],
                                          preferred_element_type=f32)
        er2t[:, pl.ds(r0, _BI)] = jnp.dot(g2, a2r_ref[...],
                                          preferred_element_type=f32).T

    @pl.when(i > _NBLK)
    def _layer2():
        r0 = (i - 1 - _NBLK) * _BI
        mask = adj_ref[...] != 0
        el2 = el2s[pl.ds(r0, _BI), :]
        er2 = er2t[...]
        p = _scores(mask, jnp.exp(el2).astype(bf16), jnp.exp(er2).astype(bf16),
                    jnp.exp(0.2 * el2).astype(bf16),
                    jnp.exp(0.2 * er2).astype(bf16))
        nd = jnp.dot(p, g2aug[...], preferred_element_type=f32)
        out_ref[...] = nd[:, :_C] / nd[:, _C:_C + 1]


@functools.partial(jax.jit, static_argnames=())
def kernel(x, adj_mat, W1, a1_l, a1_r, W2, a2_l, a2_r):
    f32 = jnp.float32
    adj = adj_mat.reshape(_N, _N).astype(jnp.int8)

    eye = jnp.eye(_H, dtype=f32)
    A1l = jnp.kron(eye, a1_l.astype(f32)[:, None])
    A1r = jnp.kron(eye, a1_r.astype(f32)[:, None])

    blkmap = lambda i: ((i + _NBLK - 1) % _NBLK, 0)
    const = lambda i: (0, 0)
    out = pl.pallas_call(
        _body,
        grid=(2 * _NBLK + 1,),
        in_specs=[
            pl.BlockSpec((_N, _F), const),
            pl.BlockSpec((_F, _H * _HD), const),
            pl.BlockSpec((_H * _HD, _H), const),
            pl.BlockSpec((_H * _HD, _H), const),
            pl.BlockSpec((_BI, _N), blkmap),
            pl.BlockSpec((_F, _C), const),
            pl.BlockSpec((_C, 1), const),
            pl.BlockSpec((_C, 1), const),
        ],
        out_specs=pl.BlockSpec((_BI, _C), blkmap),
        out_shape=jax.ShapeDtypeStruct((_N, _C), f32),
        scratch_shapes=[
            pltpu.VMEM((_N, _H * 128), jnp.bfloat16),
            pltpu.VMEM((_N, _H), f32),
            pltpu.VMEM((_H, _N), f32),
            pltpu.VMEM((_N, 64), jnp.bfloat16),
            pltpu.VMEM((_N, 1), f32),
            pltpu.VMEM((1, _N), f32),
        ],
    )(x, W1, A1l, A1r, adj, W2.astype(f32), a2_l.astype(f32)[:, None],
      a2_r.astype(f32)[:, None])
    return out


# speedup vs baseline: 1.5978x; 1.2161x over previous
import functools

import jax
import jax.numpy as jnp
from jax.experimental import pallas as pl
from jax.experimental.pallas import tpu as pltpu

_N = 2048
_H = 8
_HD = 32
_F = 256
_C = 32
_BI = 512
_NBLK = _N // _BI


def _scores(mask, ber, eneg, der):
    return jnp.where(mask, jnp.maximum(ber, eneg * der), jnp.bfloat16(0.0))


def _body(x_ref, w1_ref, al_ref, ar_ref, adj_ref, w2_ref, a2l_ref, a2r_ref,
          out_ref, gaug, el1s, er1t, g2aug, el2s, er2t):
    i = pl.program_id(0)
    f32 = jnp.float32
    bf16 = jnp.bfloat16

    @pl.when(i == 0)
    def _proj():
        g = jnp.dot(x_ref[...], w1_ref[...], preferred_element_type=f32)
        ones = jnp.ones((_N, 1), dtype=bf16)
        zeros = jnp.zeros((_N, 128 - _HD - 1), dtype=bf16)
        parts = []
        for h in range(_H):
            parts += [g[:, h * _HD:(h + 1) * _HD].astype(bf16), ones, zeros]
        gaug[...] = jnp.concatenate(parts, axis=1)
        el1s[...] = jnp.dot(g, al_ref[...], preferred_element_type=f32)
        er1t[...] = jnp.dot(g, ar_ref[...], preferred_element_type=f32).T

    @pl.when((i >= 1) & (i <= _NBLK))
    def _layer1():
        r0 = (i - 1) * _BI
        mask = adj_ref[...] != 0
        el = el1s[pl.ds(r0, _BI), :]
        ert = er1t[...]
        eneg = jnp.exp(-0.8 * el).astype(bf16)
        ber = jnp.exp(ert).astype(bf16)
        der = jnp.exp(0.2 * ert).astype(bf16)
        parts = []
        for h in range(_H):
            p = _scores(mask, ber[h:h + 1, :], eneg[:, h:h + 1],
                        der[h:h + 1, :])
            nd = jnp.dot(p, gaug[:, h * 128:(h + 1) * 128],
                         preferred_element_type=f32)
            parts.append(nd[:, :_HD] / nd[:, _HD:_HD + 1])
        hcat = jnp.concatenate(parts, axis=1)
        hact = jnp.where(hcat > 0, hcat,
                         jnp.exp(jnp.minimum(hcat, 0.0)) - 1.0)
        g2 = jnp.dot(hact, w2_ref[...], preferred_element_type=f32)
        g2aug[pl.ds(r0, _BI), :] = jnp.concatenate(
            [g2.astype(bf16), jnp.ones((_BI, 1), bf16),
             jnp.zeros((_BI, 64 - _C - 1), bf16)], axis=1)
        el2s[pl.ds(r0, _BI), :] = jnp.dot(g2, a2l_ref[...],
                                          preferred_element_type=f32)
        er2t[:, pl.ds(r0, _BI)] = jnp.dot(g2, a2r_ref[...],
                                          preferred_element_type=f32).T

    @pl.when(i > _NBLK)
    def _layer2():
        r0 = (i - 1 - _NBLK) * _BI
        mask = adj_ref[...] != 0
        el2 = el2s[pl.ds(r0, _BI), :]
        er2 = er2t[...]
        p = _scores(mask, jnp.exp(er2).astype(bf16),
                    jnp.exp(-0.8 * el2).astype(bf16),
                    jnp.exp(0.2 * er2).astype(bf16))
        nd = jnp.dot(p, g2aug[...], preferred_element_type=f32)
        out_ref[...] = nd[:, :_C] / nd[:, _C:_C + 1]


@functools.partial(jax.jit, static_argnames=())
def kernel(x, adj_mat, W1, a1_l, a1_r, W2, a2_l, a2_r):
    f32 = jnp.float32
    adj = adj_mat.reshape(_N, _N).astype(jnp.int8)

    eye = jnp.eye(_H, dtype=f32)
    A1l = jnp.kron(eye, a1_l.astype(f32)[:, None])
    A1r = jnp.kron(eye, a1_r.astype(f32)[:, None])

    blkmap = lambda i: ((i + _NBLK - 1) % _NBLK, 0)
    const = lambda i: (0, 0)
    out = pl.pallas_call(
        _body,
        grid=(2 * _NBLK + 1,),
        in_specs=[
            pl.BlockSpec((_N, _F), const),
            pl.BlockSpec((_F, _H * _HD), const),
            pl.BlockSpec((_H * _HD, _H), const),
            pl.BlockSpec((_H * _HD, _H), const),
            pl.BlockSpec((_BI, _N), blkmap),
            pl.BlockSpec((_F, _C), const),
            pl.BlockSpec((_C, 1), const),
            pl.BlockSpec((_C, 1), const),
        ],
        out_specs=pl.BlockSpec((_BI, _C), blkmap),
        out_shape=jax.ShapeDtypeStruct((_N, _C), f32),
        scratch_shapes=[
            pltpu.VMEM((_N, _H * 128), jnp.bfloat16),
            pltpu.VMEM((_N, _H), f32),
            pltpu.VMEM((_H, _N), f32),
            pltpu.VMEM((_N, 64), jnp.bfloat16),
            pltpu.VMEM((_N, 1), f32),
            pltpu.VMEM((1, _N), f32),
        ],
    )(x, W1, A1l, A1r, adj, W2.astype(f32), a2_l.astype(f32)[:, None],
      a2_r.astype(f32)[:, None])
    return out


# speedup vs baseline: 1.6922x; 1.0591x over previous
import functools

import jax
import jax.numpy as jnp
from jax.experimental import pallas as pl
from jax.experimental.pallas import tpu as pltpu

_N = 2048
_H = 8
_HD = 32
_F = 256
_C = 32
_BI = 1024
_NBLK = _N // _BI


def _scores(mask, ber, eneg, der):
    return jnp.where(mask, jnp.maximum(ber, eneg * der), jnp.bfloat16(0.0))


def _body(x_ref, w1_ref, al_ref, ar_ref, adj_ref, w2_ref, a2l_ref, a2r_ref,
          out_ref, gaug, el1s, er1t, g2aug, el2s, er2t):
    i = pl.program_id(0)
    f32 = jnp.float32
    bf16 = jnp.bfloat16

    @pl.when(i == 0)
    def _proj():
        g = jnp.dot(x_ref[...], w1_ref[...], preferred_element_type=f32)
        ones = jnp.ones((_N, 1), dtype=bf16)
        zeros = jnp.zeros((_N, 128 - _HD - 1), dtype=bf16)
        parts = []
        for h in range(_H):
            parts += [g[:, h * _HD:(h + 1) * _HD].astype(bf16), ones, zeros]
        gaug[...] = jnp.concatenate(parts, axis=1)
        el1s[...] = jnp.dot(g, al_ref[...], preferred_element_type=f32)
        er1t[...] = jnp.dot(g, ar_ref[...], preferred_element_type=f32).T

    @pl.when((i >= 1) & (i <= _NBLK))
    def _layer1():
        r0 = (i - 1) * _BI
        mask = adj_ref[...] != 0
        el = el1s[pl.ds(r0, _BI), :]
        ert = er1t[...]
        eneg = jnp.exp(-0.8 * el).astype(bf16)
        ber = jnp.exp(ert).astype(bf16)
        der = jnp.exp(0.2 * ert).astype(bf16)
        parts = []
        for h in range(_H):
            p = _scores(mask, ber[h:h + 1, :], eneg[:, h:h + 1],
                        der[h:h + 1, :])
            nd = jnp.dot(p, gaug[:, h * 128:(h + 1) * 128],
                         preferred_element_type=f32)
            parts.append(nd[:, :_HD] / nd[:, _HD:_HD + 1])
        hcat = jnp.concatenate(parts, axis=1)
        hact = jnp.where(hcat > 0, hcat,
                         jnp.exp(jnp.minimum(hcat, 0.0)) - 1.0)
        g2 = jnp.dot(hact, w2_ref[...], preferred_element_type=f32)
        g2aug[pl.ds(r0, _BI), :] = jnp.concatenate(
            [g2.astype(bf16), jnp.ones((_BI, 1), bf16),
             jnp.zeros((_BI, 64 - _C - 1), bf16)], axis=1)
        el2s[pl.ds(r0, _BI), :] = jnp.dot(g2, a2l_ref[...],
                                          preferred_element_type=f32)
        er2t[:, pl.ds(r0, _BI)] = jnp.dot(g2, a2r_ref[...],
                                          preferred_element_type=f32).T

    @pl.when(i > _NBLK)
    def _layer2():
        r0 = (i - 1 - _NBLK) * _BI
        mask = adj_ref[...] != 0
        el2 = el2s[pl.ds(r0, _BI), :]
        er2 = er2t[...]
        p = _scores(mask, jnp.exp(er2).astype(bf16),
                    jnp.exp(-0.8 * el2).astype(bf16),
                    jnp.exp(0.2 * er2).astype(bf16))
        nd = jnp.dot(p, g2aug[...], preferred_element_type=f32)
        out_ref[...] = nd[:, :_C] / nd[:, _C:_C + 1]


@functools.partial(jax.jit, static_argnames=())
def kernel(x, adj_mat, W1, a1_l, a1_r, W2, a2_l, a2_r):
    f32 = jnp.float32
    adj = adj_mat.reshape(_N, _N).astype(jnp.int8)

    eye = jnp.eye(_H, dtype=f32)
    A1l = jnp.kron(eye, a1_l.astype(f32)[:, None])
    A1r = jnp.kron(eye, a1_r.astype(f32)[:, None])

    blkmap = lambda i: ((i + _NBLK - 1) % _NBLK, 0)
    const = lambda i: (0, 0)
    out = pl.pallas_call(
        _body,
        grid=(2 * _NBLK + 1,),
        in_specs=[
            pl.BlockSpec((_N, _F), const),
            pl.BlockSpec((_F, _H * _HD), const),
            pl.BlockSpec((_H * _HD, _H), const),
            pl.BlockSpec((_H * _HD, _H), const),
            pl.BlockSpec((_BI, _N), blkmap),
            pl.BlockSpec((_F, _C), const),
            pl.BlockSpec((_C, 1), const),
            pl.BlockSpec((_C, 1), const),
        ],
        out_specs=pl.BlockSpec((_BI, _C), blkmap),
        out_shape=jax.ShapeDtypeStruct((_N, _C), f32),
        scratch_shapes=[
            pltpu.VMEM((_N, _H * 128), jnp.bfloat16),
            pltpu.VMEM((_N, _H), f32),
            pltpu.VMEM((_H, _N), f32),
            pltpu.VMEM((_N, 64), jnp.bfloat16),
            pltpu.VMEM((_N, 1), f32),
            pltpu.VMEM((1, _N), f32),
        ],
    )(x, W1, A1l, A1r, adj, W2.astype(f32), a2_l.astype(f32)[:, None],
      a2_r.astype(f32)[:, None])
    return out
